# baseline (speedup 1.0000x reference)
import jax
import jax.numpy as jnp
from jax.experimental import pallas as pl
from jax.experimental.pallas import tpu as pltpu
from jax.experimental.pallas import tpu_sc as plsc

_B, _S, _D = 4, 8192, 2048
_ST, _EN = 0, 1024
_W = _EN - _ST
_BS = 1024
_NB = _S // _BS


def _sc_gather(hid_rows, idx):
    @pl.kernel(
        out_type=jax.ShapeDtypeStruct((_B, _W), jnp.float32),
        mesh=plsc.VectorSubcoreMesh(core_axis_name="c", subcore_axis_name="s"),
        scratch_types=[
            pltpu.VMEM((1, _B), jnp.int32),
            pltpu.VMEM((_B, _W), jnp.float32),
        ],
    )
    def k(x_hbm, i_hbm, o_hbm, i_vmem, r_vmem):
        @pl.when((jax.lax.axis_index("c") == 0) & (jax.lax.axis_index("s") == 0))
        def _():
            pltpu.sync_copy(i_hbm, i_vmem)
            pltpu.sync_copy(x_hbm.at[i_vmem.at[0]], r_vmem)
            pltpu.sync_copy(r_vmem, o_hbm)

    return k(hid_rows, idx)


def _body(eot_ref, w_hbm_ref, rows_hbm_ref, hid_blk_ref, out_ref,
          w_s, rows_s, new_s, sem, wsem):
    b = pl.program_id(0)
    j = pl.program_id(1)

    out_ref[...] = hid_blk_ref[...]

    @pl.when((b == 0) & (j == 0))
    def _start_dmas():
        pltpu.make_async_copy(w_hbm_ref, w_s, wsem).start()
        pltpu.make_async_copy(rows_hbm_ref, rows_s, sem).start()

    @pl.when((b == 0) & (j == 1))
    def _rotate():
        pltpu.make_async_copy(rows_hbm_ref, rows_s, sem).wait()
        pltpu.make_async_copy(w_hbm_ref, w_s, wsem).wait()
        r = jax.lax.dot_general(
            rows_s[...], w_s[...], (((1,), (0,)), ((), ())),
            preferred_element_type=jnp.float32,
        )
        inv = jax.lax.dot_general(
            r, w_s[...], (((1,), (1,)), ((), ())),
            preferred_element_type=jnp.float32,
        )
        new_s[...] = inv.reshape(_B, 1, _W)

    @pl.when(j == _NB - 1)
    def _patch():
        local = eot_ref[b] % _BS
        out_ref[pl.ds(0, 1), pl.ds(local, 1), pl.ds(_ST, _W)] = (
            new_s[pl.ds(b, 1)]
        )


def _reorder(b, j, eot_ref):
    k_e = eot_ref[b] // _BS
    jj = jnp.where(j < k_e, j, jnp.where(j < _NB - 1, j + 1, k_e))
    return (b, jj, 0)


def kernel(hidden_states, eot_indices, W):
    eot = eot_indices.astype(jnp.int32)
    idx = ((jnp.arange(_B, dtype=jnp.int32) * _S + eot) * (_D // _W)).reshape(1, _B)
    hid_rows = hidden_states.reshape(_B * _S * (_D // _W), _W)
    rows = _sc_gather(hid_rows, idx)

    grid_spec = pltpu.PrefetchScalarGridSpec(
        num_scalar_prefetch=1,
        grid=(_B, _NB),
        in_specs=[
            pl.BlockSpec(memory_space=pltpu.MemorySpace.HBM),
            pl.BlockSpec(memory_space=pltpu.MemorySpace.HBM),
            pl.BlockSpec((1, _BS, _D), _reorder),
        ],
        out_specs=pl.BlockSpec((1, _BS, _D), _reorder),
        scratch_shapes=[
            pltpu.VMEM((_W, _W), jnp.float32),
            pltpu.VMEM((_B, _W), jnp.float32),
            pltpu.VMEM((_B, 1, _W), jnp.float32),
            pltpu.SemaphoreType.DMA,
            pltpu.SemaphoreType.DMA,
        ],
    )
    return pl.pallas_call(
        _body,
        grid_spec=grid_spec,
        out_shape=jax.ShapeDtypeStruct((_B, _S, _D), jnp.float32),
        compiler_params=pltpu.CompilerParams(
            dimension_semantics=("arbitrary", "arbitrary"),
        ),
    )(eot, W, rows, hidden_states)


# speedup vs baseline: 2.6931x; 2.6931x over previous
import jax
import jax.numpy as jnp
from jax.experimental import pallas as pl
from jax.experimental.pallas import tpu as pltpu

_B, _S, _D = 4, 8192, 2048
_ST, _EN = 0, 1024
_W = _EN - _ST
_BS = 1024
_NB = _S // _BS


def _gather_cps(eot_ref, hid_any_ref, rows_s, sem):
    cps = []
    for bb in range(_B):
        e = eot_ref[bb]
        cps.append(pltpu.make_async_copy(
            hid_any_ref.at[pl.ds(bb, 1), pl.ds(e, 1), pl.ds(_ST, _W)],
            rows_s.at[pl.ds(bb, 1)],
            sem,
        ))
    return cps


def _body(eot_ref, w_hbm_ref, hid_blk_ref, hid_any_ref, out_ref,
          w_s, rows_s, new_s, sem, wsem):
    b = pl.program_id(0)
    j = pl.program_id(1)

    out_ref[...] = hid_blk_ref[...]

    @pl.when((b == 0) & (j == 0))
    def _start_dmas():
        pltpu.make_async_copy(w_hbm_ref, w_s, wsem).start()
        for cp in _gather_cps(eot_ref, hid_any_ref, rows_s, sem):
            cp.start()

    @pl.when((b == 0) & (j == 1))
    def _rotate():
        for cp in _gather_cps(eot_ref, hid_any_ref, rows_s, sem):
            cp.wait()
        pltpu.make_async_copy(w_hbm_ref, w_s, wsem).wait()
        t = rows_s[...].reshape(_B, _W)
        r = jax.lax.dot_general(
            t, w_s[...], (((1,), (0,)), ((), ())),
            preferred_element_type=jnp.float32,
        )
        inv = jax.lax.dot_general(
            r, w_s[...], (((1,), (1,)), ((), ())),
            preferred_element_type=jnp.float32,
        )
        new_s[...] = inv.reshape(_B, 1, _W)

    @pl.when(j == _NB - 1)
    def _patch():
        local = eot_ref[b] % _BS
        out_ref[pl.ds(0, 1), pl.ds(local, 1), pl.ds(_ST, _W)] = (
            new_s[pl.ds(b, 1)]
        )


def _reorder(b, j, eot_ref):
    k_e = eot_ref[b] // _BS
    jj = jnp.where(j < k_e, j, jnp.where(j < _NB - 1, j + 1, k_e))
    return (b, jj, 0)


def kernel(hidden_states, eot_indices, W):
    eot = eot_indices.astype(jnp.int32)
    grid_spec = pltpu.PrefetchScalarGridSpec(
        num_scalar_prefetch=1,
        grid=(_B, _NB),
        in_specs=[
            pl.BlockSpec(memory_space=pltpu.MemorySpace.HBM),
            pl.BlockSpec((1, _BS, _D), _reorder),
            pl.BlockSpec(memory_space=pltpu.MemorySpace.HBM),
        ],
        out_specs=pl.BlockSpec((1, _BS, _D), _reorder),
        scratch_shapes=[
            pltpu.VMEM((_W, _W), jnp.float32),
            pltpu.VMEM((_B, 1, _W), jnp.float32),
            pltpu.VMEM((_B, 1, _W), jnp.float32),
            pltpu.SemaphoreType.DMA,
            pltpu.SemaphoreType.DMA,
        ],
    )
    return pl.pallas_call(
        _body,
        grid_spec=grid_spec,
        out_shape=jax.ShapeDtypeStruct((_B, _S, _D), jnp.float32),
        compiler_params=pltpu.CompilerParams(
            dimension_semantics=("arbitrary", "arbitrary"),
        ),
    )(eot, W, hidden_states, hidden_states)
